# Initial kernel scaffold; baseline (speedup 1.0000x reference)
import jax
import jax.numpy as jnp
from jax.experimental import pallas as pl


def kernel(pts, params):
    raise NotImplementedError("write your pallas kernel here")


# speedup vs baseline: 4.1035x; 4.1035x over previous
import functools
import math

import jax
import jax.numpy as jnp
from jax import lax
from jax.experimental import pallas as pl
from jax.experimental.pallas import tpu as pltpu
from jax.experimental.pallas import tpu_sc as plsc

_D = 384
_K = 16
_NEG = -1e30


def _dotT(a, w):
    return lax.dot_general(a, w, (((1,), (1,)), ((), ())),
                           precision=lax.Precision.HIGHEST,
                           preferred_element_type=jnp.float32)


def _dot(a, w):
    return lax.dot_general(a, w, (((1,), (0,)), ((), ())),
                           precision=lax.Precision.HIGHEST,
                           preferred_element_type=jnp.float32)


def _lin_body(x_ref, pos_ref, wsrc_ref, wval_ref, wdst_ref, tab_ref, adst_ref):
    x = x_ref[0]
    tab_ref[0, :, 0:384] = _dotT(x, wsrc_ref[...])
    tab_ref[0, :, 384:768] = _dotT(x, wval_ref[...])
    tab_ref[0, :, 768:896] = pos_ref[0]
    adst_ref[0] = _dotT(x, wdst_ref[...])


def _lin_stage(x, pos128, wsrc, wval, wdst):
    b, n, _ = x.shape
    tn = min(n, 256)
    grid = (b, n // tn)
    full = lambda i, j: (0, 0)
    tab, adst = pl.pallas_call(
        _lin_body,
        grid=grid,
        in_specs=[
            pl.BlockSpec((1, tn, _D), lambda i, j: (i, j, 0)),
            pl.BlockSpec((1, tn, 128), lambda i, j: (i, j, 0)),
            pl.BlockSpec((_D, _D), full),
            pl.BlockSpec((_D, _D), full),
            pl.BlockSpec((_D, _D), full),
        ],
        out_specs=[
            pl.BlockSpec((1, tn, 896), lambda i, j: (i, j, 0)),
            pl.BlockSpec((1, tn, _D), lambda i, j: (i, j, 0)),
        ],
        out_shape=[
            jax.ShapeDtypeStruct((b, n, 896), jnp.float32),
            jax.ShapeDtypeStruct((b, n, _D), jnp.float32),
        ],
    )(x, pos128, wsrc, wval, wdst)
    return tab, adst


def _fps_body(posL_ref, out_ref, *, b, n, sub, m):
    nl = n // sub
    xr = posL_ref[0]
    yr = posL_ref[1]
    zr = posL_ref[2]
    li = (lax.broadcasted_iota(jnp.int32, (b, sub, nl), 1) * nl
          + lax.broadcasted_iota(jnp.int32, (b, sub, nl), 2))
    lim = lax.broadcasted_iota(jnp.int32, (b, m), 1)

    def _rsum(v):
        return jnp.sum(jnp.sum(v, axis=2, keepdims=True), axis=1, keepdims=True)

    def _rmax(v):
        return jnp.max(jnp.max(v, axis=2, keepdims=True), axis=1, keepdims=True)

    def _rmin(v):
        return jnp.min(jnp.min(v, axis=2, keepdims=True), axis=1, keepdims=True)

    def body(t, carry):
        dist, last3, idxs = carry
        idxs = jnp.where(lim == t, last3[:, :, 0], idxs)
        oh = li == last3
        xl = _rsum(jnp.where(oh, xr, 0.0))
        yl = _rsum(jnp.where(oh, yr, 0.0))
        zl = _rsum(jnp.where(oh, zr, 0.0))
        dx = xr - xl
        dy = yr - yl
        dz = zr - zl
        d = dx * dx + dy * dy + dz * dz
        dist = jnp.minimum(dist, d)
        mx = _rmax(dist)
        cand = jnp.where(dist == mx, li, n)
        last3 = _rmin(cand)
        return dist, last3, idxs

    dist0 = jnp.full((b, sub, nl), jnp.inf, jnp.float32)
    last0 = jnp.zeros((b, 1, 1), jnp.int32)
    idxs0 = jnp.zeros((b, m), jnp.int32)
    _, _, idxs = lax.fori_loop(0, m, body, (dist0, last0, idxs0))
    out_ref[...] = idxs


def _fps_stage(posL, m):
    three, b, sub, nl = posL.shape
    n = sub * nl
    return pl.pallas_call(
        functools.partial(_fps_body, b=b, n=n, sub=sub, m=m),
        out_shape=jax.ShapeDtypeStruct((b, m), jnp.int32),
    )(posL)


def _nbr_body(idxc_ref, posR_ref, pospad_ref, adst_ref,
              nbrg_ref, qpos_ref, qadst_ref, *, n, tm):
    bb = pl.program_id(0)
    qidx = idxc_ref[0]
    li = lax.broadcasted_iota(jnp.int32, (tm, n), 1)
    ohf = (qidx == li).astype(jnp.float32)
    qpos = _dot(ohf, pospad_ref[0])
    qadst = _dot(ohf, adst_ref[0])

    xr = posR_ref[0, 0:1, :]
    yr = posR_ref[0, 1:2, :]
    zr = posR_ref[0, 2:3, :]
    dx = qpos[:, 0:1] - xr
    dy = qpos[:, 1:2] - yr
    dz = qpos[:, 2:3] - zr
    d2 = dx * dx + dy * dy + dz * dz

    cur = d2
    cols = []
    for _ in range(_K):
        mn = jnp.min(cur, axis=1, keepdims=True)
        am = jnp.min(jnp.where(cur == mn, li, n), axis=1, keepdims=True)
        cols.append(am)
        cur = jnp.where(li == am, jnp.inf, cur)
    nbr = jnp.concatenate(cols, axis=1)
    nbrg_ref[0] = nbr + bb * n
    qpos_ref[0] = qpos
    qadst_ref[0] = qadst


def _nbr_stage(idxcol, posR, pospad, adst):
    b, m, _ = idxcol.shape
    n = posR.shape[2]
    tm = min(m, 128)
    grid = (b, m // tm)
    nbrg, qpos, qadst = pl.pallas_call(
        functools.partial(_nbr_body, n=n, tm=tm),
        grid=grid,
        in_specs=[
            pl.BlockSpec((1, tm, 1), lambda i, j: (i, j, 0)),
            pl.BlockSpec((1, 8, n), lambda i, j: (i, 0, 0)),
            pl.BlockSpec((1, n, 16), lambda i, j: (i, 0, 0)),
            pl.BlockSpec((1, n, _D), lambda i, j: (i, 0, 0)),
        ],
        out_specs=[
            pl.BlockSpec((1, tm, _K), lambda i, j: (i, j, 0)),
            pl.BlockSpec((1, tm, 16), lambda i, j: (i, j, 0)),
            pl.BlockSpec((1, tm, _D), lambda i, j: (i, j, 0)),
        ],
        out_shape=[
            jax.ShapeDtypeStruct((b, m, _K), jnp.int32),
            jax.ShapeDtypeStruct((b, m, 16), jnp.float32),
            jax.ShapeDtypeStruct((b, m, _D), jnp.float32),
        ],
    )(idxcol, posR, pospad, adst)
    return nbrg, qpos, qadst


def _gather_rows(table, gidx):
    g = gidx.shape[0]
    c = table.shape[1]
    info = plsc.get_sparse_core_info()
    nc, ns = info.num_cores, info.num_subcores
    nw = nc * ns
    per_w = g // nw
    ch = min(64, per_w)
    reps = per_w // ch
    mesh = plsc.VectorSubcoreMesh(core_axis_name="c", subcore_axis_name="s")

    @functools.partial(
        pl.kernel, mesh=mesh,
        out_type=jax.ShapeDtypeStruct((g, c), jnp.float32),
        scratch_types=[
            pltpu.VMEM((ch,), jnp.int32),
            pltpu.VMEM((ch, c), jnp.float32),
            pltpu.SemaphoreType.DMA,
        ],
    )
    def gk(idx_hbm, tab_hbm, out_hbm, idx_v, rows_v, sem):
        wid = lax.axis_index("s") * nc + lax.axis_index("c")
        for j in range(reps):
            base = wid * per_w + j * ch
            pltpu.sync_copy(idx_hbm.at[pl.ds(base, ch)], idx_v)
            pltpu.async_copy(tab_hbm.at[idx_v], rows_v, sem).wait()
            pltpu.sync_copy(rows_v, out_hbm.at[pl.ds(base, ch)])

    return gk(gidx, table)


def _attn_body(ge_ref, qadst_ref, qpos_ref, w1p_ref, b1_ref, w2_ref, b2_ref,
               wa_ref, ba_ref, g_ref, be_ref, out_ref, *, tq, r2):
    tqk = tq * _K
    ge = ge_ref[0]
    a_src = ge[:, 0:384]
    val = ge[:, 384:768]
    pose = ge[:, 768:784]
    qpos = qpos_ref[0]
    qadst = qadst_ref[0]

    qpos_e = jnp.reshape(
        jnp.broadcast_to(qpos[:, None, :], (tq, _K, 16)), (tqk, 16))
    rel = qpos_e - pose
    d2 = jnp.sum(rel * rel, axis=1, keepdims=True)
    validf = d2 <= r2

    h = jax.nn.relu(_dotT(rel, w1p_ref[...]) + b1_ref[...])
    delta = jax.nn.relu(_dotT(h, w2_ref[...]) + b2_ref[...])

    adst_e = jnp.reshape(
        jnp.broadcast_to(qadst[:, None, :], (tq, _K, _D)), (tqk, _D))
    ae = adst_e - a_src + delta
    s = jax.nn.relu(_dotT(ae, wa_ref[...]) + ba_ref[...])
    s = jnp.where(validf, s, _NEG)

    s3 = jnp.reshape(s, (tq, _K, _D))
    mx = jnp.max(s3, axis=1, keepdims=True)
    e = jnp.exp(s3 - mx)
    sm = e / jnp.sum(e, axis=1, keepdims=True)
    msg = sm * jnp.reshape(val + delta, (tq, _K, _D))
    msg = jnp.where(jnp.reshape(validf, (tq, _K, 1)), msg, 0.0)
    o = jnp.sum(msg, axis=1)

    mu = jnp.mean(o, axis=1, keepdims=True)
    var = jnp.mean((o - mu) * (o - mu), axis=1, keepdims=True)
    o = (o - mu) * lax.rsqrt(var + 1e-5) * g_ref[...] + be_ref[...]
    out_ref[0] = o


def _attn_stage(ge, qadst, qpos, w1p, b1, w2, b2, wa, ba, g, be, r):
    b, m, _ = qadst.shape
    tq = min(m, 64)
    grid = (b, m // tq)
    full = lambda i, j: (0, 0)
    out = pl.pallas_call(
        functools.partial(_attn_body, tq=tq, r2=float(r) * float(r)),
        grid=grid,
        in_specs=[
            pl.BlockSpec((1, tq * _K, 896), lambda i, j: (i, j, 0)),
            pl.BlockSpec((1, tq, _D), lambda i, j: (i, j, 0)),
            pl.BlockSpec((1, tq, 16), lambda i, j: (i, j, 0)),
            pl.BlockSpec((_D, 16), full),
            pl.BlockSpec((1, _D), full),
            pl.BlockSpec((_D, _D), full),
            pl.BlockSpec((1, _D), full),
            pl.BlockSpec((_D, _D), full),
            pl.BlockSpec((1, _D), full),
            pl.BlockSpec((1, _D), full),
            pl.BlockSpec((1, _D), full),
        ],
        out_specs=pl.BlockSpec((1, tq, _D), lambda i, j: (i, j, 0)),
        out_shape=jax.ShapeDtypeStruct((b, m, _D), jnp.float32),
    )(ge, qadst, qpos, w1p, b1, w2, b2, wa, ba, g, be)
    return out


def _proj_body(x_ref, w_ref, b_ref, g_ref, be_ref, out_ref):
    o = _dotT(x_ref[...], w_ref[...]) + b_ref[...]
    mu = jnp.mean(o, axis=1, keepdims=True)
    var = jnp.mean((o - mu) * (o - mu), axis=1, keepdims=True)
    out_ref[...] = (o - mu) * lax.rsqrt(var + 1e-5) * g_ref[...] + be_ref[...]


def _proj_stage(x2d, w, bias, g, be):
    r = x2d.shape[0]
    return pl.pallas_call(
        _proj_body,
        out_shape=jax.ShapeDtypeStruct((r, _D), jnp.float32),
    )(x2d, w, bias, g, be)


def _row(v):
    return jnp.reshape(v, (1, _D))


def _block(p, x, pospad, r):
    b, n, _ = x.shape
    m = int(math.ceil(0.25 * n))
    sub = 16 if n % 16 == 0 else 8
    pos128 = jnp.pad(pospad, ((0, 0), (0, 0), (0, 112)))
    posR = jnp.transpose(pospad, (0, 2, 1))[:, :8, :]
    posL = jnp.reshape(jnp.transpose(pospad[:, :, :3], (2, 0, 1)),
                       (3, b, sub, n // sub))

    tab, adst = _lin_stage(x, pos128, p['lin_src'], p['lin'], p['lin_dst'])
    idx = _fps_stage(posL, m)
    idxcol = jnp.reshape(idx, (b, m, 1))
    nbrg, qpos, qadst = _nbr_stage(idxcol, posR, pospad, adst)

    glist = jnp.reshape(nbrg, (b * m * _K,))
    ge = _gather_rows(jnp.reshape(tab, (b * n, 896)), glist)
    ge = jnp.reshape(ge, (b, m * _K, 896))

    w1p = jnp.pad(p['pos_w1'], ((0, 0), (0, 13)))
    x2 = _attn_stage(ge, qadst, qpos, w1p, _row(p['pos_b1']),
                     p['pos_w2'], _row(p['pos_b2']),
                     p['attn_w'], _row(p['attn_b']),
                     _row(p['ln_g']), _row(p['ln_b']), r)
    return x2, qpos


def kernel(pts, params):
    b, n, _ = pts.shape
    xyz = pts[..., :3]
    x = pts[..., 3:]
    pospad = jnp.pad(xyz, ((0, 0), (0, 0), (0, 13)))

    radii = (1.0, 2.0, 4.0, 8.0)
    for i in range(4):
        x, pospad = _block(params['sa%d' % (i + 1)], x, pospad, radii[i])

    m = x.shape[1]
    f2d = _proj_stage(jnp.reshape(x, (b * m, _D)), params['proj_w'],
                      _row(params['proj_b']), _row(params['proj_ln_g']),
                      _row(params['proj_ln_b']))
    f = jnp.reshape(f2d, (b, m, _D))
    pos_out = pospad[:, :, :3]
    mask = jnp.zeros((b, m), dtype=bool)
    return pos_out, f, mask
